# baseline (speedup 1.0000x reference)
import functools

import jax
import jax.numpy as jnp
import numpy as np
from jax import lax
from jax.experimental import pallas as pl
from jax.experimental.pallas import tpu as pltpu
from jax.experimental.pallas import tpu_sc as plsc

N = 10000
E = 160000
D = 16
EED = 16
EHID = 64
STEPS = 3
G = 64

NC = 2
NS = 16
NW = NC * NS
EH = E // 2
EPW = EH // NW
CH = 125
NCHUNK = EPW // CH
KFIRE = 10
NPS = N // NS

_SC_MESH = plsc.VectorSubcoreMesh(
    core_axis_name="c", subcore_axis_name="s", num_cores=NC, num_subcores=NS)
_SC_PARAMS = pltpu.CompilerParams(use_tc_tiling_on_sc=False)


@functools.partial(
    pl.kernel,
    out_type=jax.ShapeDtypeStruct((EH, D), jnp.float32),
    mesh=_SC_MESH,
    compiler_params=_SC_PARAMS,
    scratch_types=[
        pltpu.VMEM((NCHUNK, CH), jnp.int32),
        pltpu.VMEM((EPW, D), jnp.float32),
        pltpu.SemaphoreType.DMA,
    ],
)
def _sc_gather(h_hbm, src_hbm, out_hbm, idx_v, rows_v, sem):
    wid = lax.axis_index("s") * NC + lax.axis_index("c")
    pltpu.sync_copy(src_hbm.at[pl.ds(wid * NCHUNK, NCHUNK)], idx_v)

    def group(go, carry):
        handles = []
        for b in range(KFIRE):
            g = go * KFIRE + b
            handles.append(pltpu.async_copy(
                h_hbm.at[idx_v.at[g]], rows_v.at[pl.ds(g * CH, CH)], sem))
        for h in handles:
            h.wait()
        return carry

    lax.fori_loop(0, NCHUNK // KFIRE, group, 0)
    pltpu.sync_copy(rows_v, out_hbm.at[pl.ds(wid * EPW, EPW)])


@functools.partial(
    pl.kernel,
    out_type=jax.ShapeDtypeStruct((NC, N, D), jnp.float32),
    mesh=_SC_MESH,
    compiler_params=_SC_PARAMS,
    scratch_types=[
        pltpu.VMEM((NCHUNK, CH), jnp.int32),
        pltpu.VMEM((EPW, D), jnp.float32),
        pltpu.VMEM_SHARED((N, D), jnp.float32),
        pltpu.SemaphoreType.DMA,
    ],
)
def _sc_scatter(msg_hbm, dst_hbm, zeros_hbm, out_hbm, idx_v, rows_v, acc_sh, sem):
    c = lax.axis_index("c")
    s = lax.axis_index("s")
    wid = s * NC + c
    pltpu.sync_copy(zeros_hbm.at[pl.ds(s * NPS, NPS)], acc_sh.at[pl.ds(s * NPS, NPS)])
    pltpu.sync_copy(dst_hbm.at[pl.ds(wid * NCHUNK, NCHUNK)], idx_v)
    pltpu.sync_copy(msg_hbm.at[pl.ds(wid * EPW, EPW)], rows_v)
    plsc.subcore_barrier()

    def group(go, carry):
        handles = []
        for b in range(KFIRE):
            g = go * KFIRE + b
            handles.append(pltpu.async_copy(
                rows_v.at[pl.ds(g * CH, CH)], acc_sh.at[idx_v.at[g]], sem, add=True))
        for h in handles:
            h.wait()
        return carry

    lax.fori_loop(0, NCHUNK // KFIRE, group, 0)
    plsc.subcore_barrier()
    pltpu.sync_copy(acc_sh.at[pl.ds(s * NPS, NPS)], out_hbm.at[c, pl.ds(s * NPS, NPS)])


def _full(shape):
    return pl.BlockSpec(shape, lambda i: (0,) * len(shape))


def _node_emb(x, w, b):
    TB = 2000

    def body(x_ref, w_ref, b_ref, o_ref):
        o_ref[...] = jnp.dot(x_ref[...], w_ref[...],
                             preferred_element_type=jnp.float32) + b_ref[...]

    return pl.pallas_call(
        body,
        grid=(N // TB,),
        in_specs=[
            pl.BlockSpec((TB, 128), lambda i: (i, 0)),
            _full((128, D)),
            _full((1, D)),
        ],
        out_specs=pl.BlockSpec((TB, D), lambda i: (i, 0)),
        out_shape=jax.ShapeDtypeStruct((N, D), jnp.float32),
    )(x, w, b)


def _msg(edge_attr, xj, w1, b1, w2, b2, w3, B3m, Rm, Sm):
    TB = 3200

    def body(ea_ref, xj_ref, w1_ref, b1_ref, w2_ref, b2_ref,
             w3_ref, B3_ref, R_ref, S_ref, o_ref):
        f32 = jnp.float32
        bf = jnp.bfloat16
        TBP = TB // 8
        eaw = ea_ref[...]
        xjw = xj_ref[...]
        eav = jnp.concatenate([eaw[:, j * EED:(j + 1) * EED] for j in range(8)], axis=0)
        xjv = jnp.concatenate([xjw[:, j * D:(j + 1) * D] for j in range(8)], axis=0)
        h1 = jnp.maximum(jnp.dot(eav, w1_ref[...], preferred_element_type=f32)
                         + b1_ref[...], 0.0).astype(bf)
        h2 = jnp.maximum(jnp.dot(h1, w2_ref[...], preferred_element_type=f32)
                         + b2_ref[...], 0.0).astype(bf)
        ee = jnp.dot(h2, w3_ref[...], preferred_element_type=f32).astype(bf)
        xb = xjv.astype(bf)
        xr = jnp.dot(xb, R_ref[...], preferred_element_type=f32).astype(bf)
        msg = (jnp.dot(ee * xr, S_ref[...], preferred_element_type=f32)
               + jnp.dot(xb, B3_ref[...], preferred_element_type=f32))
        o_ref[...] = jnp.concatenate(
            [msg[j * TBP:(j + 1) * TBP, :] for j in range(8)], axis=1)

    return pl.pallas_call(
        body,
        grid=(E // TB,),
        in_specs=[
            pl.BlockSpec((TB // 8, 8 * EED), lambda i: (i, 0)),
            pl.BlockSpec((TB // 8, 8 * D), lambda i: (i, 0)),
            _full((EED, EHID)), _full((1, EHID)),
            _full((EHID, EHID)), _full((1, EHID)),
            _full((EHID, D * D)), _full((D, D)),
            _full((D, D * D)), _full((D * D, D)),
        ],
        out_specs=pl.BlockSpec((TB // 8, 8 * D), lambda i: (i, 0)),
        out_shape=jax.ShapeDtypeStruct((EH // 8, 8 * D), jnp.float32),
    )(edge_attr, xj, w1, b1, w2, b2, w3, B3m, Rm, Sm)


def _gru(parts, parts2, h, wir, wiz, win, whr, whz, whn, bir, biz, bin_, bhr, bhz, bhn):
    TB = 2000

    def body(p_ref, q_ref, h_ref, wir_ref, wiz_ref, win_ref, whr_ref, whz_ref, whn_ref,
             bir_ref, biz_ref, bin_ref, bhr_ref, bhz_ref, bhn_ref, o_ref):
        f32 = jnp.float32
        aggr = (p_ref[0] + p_ref[1]) + (q_ref[0] + q_ref[1])
        hv = h_ref[...]
        r = jax.nn.sigmoid(jnp.dot(aggr, wir_ref[...], preferred_element_type=f32) + bir_ref[...]
                           + jnp.dot(hv, whr_ref[...], preferred_element_type=f32) + bhr_ref[...])
        z = jax.nn.sigmoid(jnp.dot(aggr, wiz_ref[...], preferred_element_type=f32) + biz_ref[...]
                           + jnp.dot(hv, whz_ref[...], preferred_element_type=f32) + bhz_ref[...])
        n = jnp.tanh(jnp.dot(aggr, win_ref[...], preferred_element_type=f32) + bin_ref[...]
                     + r * (jnp.dot(hv, whn_ref[...], preferred_element_type=f32) + bhn_ref[...]))
        o_ref[...] = (1.0 - z) * n + z * hv

    wspec = _full((D, D))
    bspec = _full((1, D))
    return pl.pallas_call(
        body,
        grid=(N // TB,),
        in_specs=[
            pl.BlockSpec((NC, TB, D), lambda i: (0, i, 0)),
            pl.BlockSpec((NC, TB, D), lambda i: (0, i, 0)),
            pl.BlockSpec((TB, D), lambda i: (i, 0)),
            wspec, wspec, wspec, wspec, wspec, wspec,
            bspec, bspec, bspec, bspec, bspec, bspec,
        ],
        out_specs=pl.BlockSpec((TB, D), lambda i: (i, 0)),
        out_shape=jax.ShapeDtypeStruct((N, D), jnp.float32),
    )(parts, parts2, h, wir, wiz, win, whr, whz, whn, bir, biz, bin_, bhr, bhz, bhn)


def _readout(h, node, batch2, wi0, bi0, wi1, bi1, wi2, bi2,
             wj0, bj0, wj1, bj1, wj2, bj2, wo0, bo0, wo1, bo1):
    TB = 2000

    def body(h_ref, node_ref, batch_ref, wi0_ref, bi0_ref, wi1_ref, bi1_ref,
             wi2_ref, bi2_ref, wj0_ref, bj0_ref, wj1_ref, bj1_ref, wj2_ref,
             bj2_ref, wo0_ref, bo0_ref, wo1_ref, bo1_ref, o_ref, acc_ref):
        f32 = jnp.float32
        i = pl.program_id(0)

        @pl.when(i == 0)
        def _init():
            acc_ref[...] = jnp.zeros_like(acc_ref)

        hv = h_ref[...]
        hf = jnp.concatenate([hv, node_ref[...]], axis=1)
        a = jnp.maximum(jnp.dot(hf, wi0_ref[...], preferred_element_type=f32) + bi0_ref[...], 0.0)
        a = jnp.maximum(jnp.dot(a, wi1_ref[...], preferred_element_type=f32) + bi1_ref[...], 0.0)
        logits = jnp.dot(a, wi2_ref[...], preferred_element_type=f32) + bi2_ref[...]
        m = jnp.max(logits, axis=1, keepdims=True)
        ex = jnp.exp(logits - m)
        act = ex / jnp.sum(ex, axis=1, keepdims=True)
        b = jnp.maximum(jnp.dot(hv, wj0_ref[...], preferred_element_type=f32) + bj0_ref[...], 0.0)
        b = jnp.maximum(jnp.dot(b, wj1_ref[...], preferred_element_type=f32) + bj1_ref[...], 0.0)
        jv = jnp.dot(b, wj2_ref[...], preferred_element_type=f32) + bj2_ref[...]
        mult = act * jv
        mask = (batch_ref[...] == lax.broadcasted_iota(jnp.int32, (TB, G), 1)).astype(f32)
        acc_ref[...] += lax.dot_general(mask, mult, (((0,), (0,)), ((), ())),
                                        preferred_element_type=f32)

        @pl.when(i == pl.num_programs(0) - 1)
        def _fin():
            Racc = acc_ref[...]
            o = jnp.maximum(jnp.dot(Racc, wo0_ref[...], preferred_element_type=f32)
                            + bo0_ref[...], 0.0)
            o = jnp.dot(o, wo1_ref[...], preferred_element_type=f32) + bo1_ref[...]
            mo = jnp.max(o, axis=1, keepdims=True)
            lse = mo + jnp.log(jnp.sum(jnp.exp(o - mo), axis=1, keepdims=True))
            o_ref[...] = o - lse

    return pl.pallas_call(
        body,
        grid=(N // TB,),
        in_specs=[
            pl.BlockSpec((TB, D), lambda i: (i, 0)),
            pl.BlockSpec((TB, D), lambda i: (i, 0)),
            pl.BlockSpec((TB, 1), lambda i: (i, 0)),
            _full((2 * D, 2 * D)), _full((1, 2 * D)),
            _full((2 * D, 2 * D)), _full((1, 2 * D)),
            _full((2 * D, D)), _full((1, D)),
            _full((D, 2 * D)), _full((1, 2 * D)),
            _full((2 * D, 2 * D)), _full((1, 2 * D)),
            _full((2 * D, D)), _full((1, D)),
            _full((D, 2 * D)), _full((1, 2 * D)),
            _full((2 * D, 2)), _full((1, 2)),
        ],
        out_specs=pl.BlockSpec((G, 2), lambda i: (0, 0)),
        out_shape=jax.ShapeDtypeStruct((G, 2), jnp.float32),
        scratch_shapes=[pltpu.VMEM((G, D), jnp.float32)],
    )(h, node, batch2, wi0, bi0, wi1, bi1, wi2, bi2,
      wj0, bj0, wj1, bj1, wj2, bj2, wo0, bo0, wo1, bo1)


_RM = np.kron(np.ones((1, D), np.float32), np.eye(D, dtype=np.float32))
_SM = np.kron(np.eye(D, dtype=np.float32), np.ones((D, 1), np.float32))


def kernel(x, edge_attr, edge_index, batch, params):
    NR = EH // CH
    src2 = edge_index[0].reshape(E // CH, CH)
    dst2 = edge_index[1].reshape(E // CH, CH)
    srcA, srcB = src2[:NR], src2[NR:]
    dstA, dstB = dst2[:NR], dst2[NR:]
    batch2 = batch.reshape(N, 1)
    zeros_nd = jnp.zeros((N, D), jnp.float32)
    Rm = jnp.asarray(_RM, jnp.bfloat16)
    Sm = jnp.asarray(_SM, jnp.bfloat16)
    ea16 = edge_attr.astype(jnp.bfloat16).reshape(E // 8, 8 * EED)
    eaA, eaB = ea16[:EH // 8], ea16[EH // 8:]

    node = _node_emb(x, params["node_emb"]["w"], params["node_emb"]["b"].reshape(1, D))
    h = node
    we = params["edge_emb"]["w"]
    be = params["edge_emb"]["b"]
    bf = jnp.bfloat16
    for t in range(STEPS):
        lay = params["layers"][t]
        en = lay["edge_net"]
        w1f = (we @ en[0]["w"]).astype(bf)
        b1f = (be @ en[0]["w"] + en[0]["b"]).reshape(1, -1)
        w2f = en[1]["w"].astype(bf)
        b2f = en[1]["b"].reshape(1, -1).astype(bf)
        w3f = en[2]["w"].astype(bf)
        B3m = en[2]["b"].reshape(D, D).T.astype(bf)
        xjA = _sc_gather(h, srcA)
        xjB = _sc_gather(h, srcB)
        msgA = _msg(eaA, xjA.reshape(EH // 8, 8 * D),
                    w1f, b1f, w2f, b2f, w3f, B3m, Rm, Sm)
        partsA = _sc_scatter(msgA.reshape(EH, D), dstA, zeros_nd)
        msgB = _msg(eaB, xjB.reshape(EH // 8, 8 * D),
                    w1f, b1f, w2f, b2f, w3f, B3m, Rm, Sm)
        partsB = _sc_scatter(msgB.reshape(EH, D), dstB, zeros_nd)
        gru = lay["gru"]
        wi, wh, bi, bh = gru["wi"], gru["wh"], gru["bi"], gru["bh"]
        h = _gru(partsA, partsB, h,
                 wi[:D].T, wi[D:2 * D].T, wi[2 * D:].T,
                 wh[:D].T, wh[D:2 * D].T, wh[2 * D:].T,
                 bi[:D].reshape(1, D), bi[D:2 * D].reshape(1, D), bi[2 * D:].reshape(1, D),
                 bh[:D].reshape(1, D), bh[D:2 * D].reshape(1, D), bh[2 * D:].reshape(1, D))

    pi, pj, po = params["i"], params["j"], params["out"]
    return _readout(
        h, node, batch2,
        pi[0]["w"], pi[0]["b"].reshape(1, -1),
        pi[1]["w"], pi[1]["b"].reshape(1, -1),
        pi[2]["w"], pi[2]["b"].reshape(1, -1),
        pj[0]["w"], pj[0]["b"].reshape(1, -1),
        pj[1]["w"], pj[1]["b"].reshape(1, -1),
        pj[2]["w"], pj[2]["b"].reshape(1, -1),
        po[0]["w"], po[0]["b"].reshape(1, -1),
        po[1]["w"], po[1]["b"].reshape(1, -1))


# speedup vs baseline: 1.7295x; 1.7295x over previous
import functools

import jax
import jax.numpy as jnp
import numpy as np
from jax import lax
from jax.experimental import pallas as pl
from jax.experimental.pallas import tpu as pltpu
from jax.experimental.pallas import tpu_sc as plsc

N = 10000
E = 160000
D = 16
EED = 16
EHID = 64
STEPS = 3
G = 64

NC = 2
NS = 16
NW = NC * NS
EPW = E // NW
CH = 125
NCHUNK = EPW // CH
KFIRE = 20
NPS = N // NS

_SC_MESH = plsc.VectorSubcoreMesh(
    core_axis_name="c", subcore_axis_name="s", num_cores=NC, num_subcores=NS)
_SC_PARAMS = pltpu.CompilerParams(use_tc_tiling_on_sc=False)


@functools.partial(
    pl.kernel,
    out_type=jax.ShapeDtypeStruct((E, D), jnp.float32),
    mesh=_SC_MESH,
    compiler_params=_SC_PARAMS,
    scratch_types=[
        pltpu.VMEM((NCHUNK, CH), jnp.int32),
        pltpu.VMEM((EPW, D), jnp.float32),
        pltpu.SemaphoreType.DMA,
    ],
)
def _sc_gather(h_hbm, src_hbm, out_hbm, idx_v, rows_v, sem):
    wid = lax.axis_index("s") * NC + lax.axis_index("c")
    pltpu.sync_copy(src_hbm.at[pl.ds(wid * NCHUNK, NCHUNK)], idx_v)

    def group(go, carry):
        handles = []
        for b in range(KFIRE):
            g = go * KFIRE + b
            handles.append(pltpu.async_copy(
                h_hbm.at[idx_v.at[g]], rows_v.at[pl.ds(g * CH, CH)], sem))
        for h in handles:
            h.wait()
        return carry

    lax.fori_loop(0, NCHUNK // KFIRE, group, 0)
    pltpu.sync_copy(rows_v, out_hbm.at[pl.ds(wid * EPW, EPW)])


@functools.partial(
    pl.kernel,
    out_type=jax.ShapeDtypeStruct((NC, N, D), jnp.float32),
    mesh=_SC_MESH,
    compiler_params=_SC_PARAMS,
    scratch_types=[
        pltpu.VMEM((NCHUNK, CH), jnp.int32),
        pltpu.VMEM((EPW, D), jnp.float32),
        pltpu.VMEM_SHARED((N, D), jnp.float32),
        pltpu.SemaphoreType.DMA,
    ],
)
def _sc_scatter(msg_hbm, dst_hbm, zeros_hbm, out_hbm, idx_v, rows_v, acc_sh, sem):
    c = lax.axis_index("c")
    s = lax.axis_index("s")
    wid = s * NC + c
    pltpu.sync_copy(zeros_hbm.at[pl.ds(s * NPS, NPS)], acc_sh.at[pl.ds(s * NPS, NPS)])
    pltpu.sync_copy(dst_hbm.at[pl.ds(wid * NCHUNK, NCHUNK)], idx_v)
    pltpu.sync_copy(msg_hbm.at[pl.ds(wid * EPW, EPW)], rows_v)
    plsc.subcore_barrier()

    def group(go, carry):
        handles = []
        for b in range(KFIRE):
            g = go * KFIRE + b
            handles.append(pltpu.async_copy(
                rows_v.at[pl.ds(g * CH, CH)], acc_sh.at[idx_v.at[g]], sem, add=True))
        for h in handles:
            h.wait()
        return carry

    lax.fori_loop(0, NCHUNK // KFIRE, group, 0)
    plsc.subcore_barrier()
    pltpu.sync_copy(acc_sh.at[pl.ds(s * NPS, NPS)], out_hbm.at[c, pl.ds(s * NPS, NPS)])


def _full(shape):
    return pl.BlockSpec(shape, lambda i: (0,) * len(shape))


def _node_emb(x, w, b):
    TB = 2000

    def body(x_ref, w_ref, b_ref, o_ref):
        o_ref[...] = jnp.dot(x_ref[...], w_ref[...],
                             preferred_element_type=jnp.float32) + b_ref[...]

    return pl.pallas_call(
        body,
        grid=(N // TB,),
        in_specs=[
            pl.BlockSpec((TB, 128), lambda i: (i, 0)),
            _full((128, D)),
            _full((1, D)),
        ],
        out_specs=pl.BlockSpec((TB, D), lambda i: (i, 0)),
        out_shape=jax.ShapeDtypeStruct((N, D), jnp.float32),
    )(x, w, b)


def _msg(edge_attr, xj, w1, b1, w2, b2, w3, B3m, Rm, Sm):
    TB = 6400

    def body(ea_ref, xj_ref, w1_ref, b1_ref, w2_ref, b2_ref,
             w3_ref, B3_ref, R_ref, S_ref, o_ref):
        f32 = jnp.float32
        bf = jnp.bfloat16
        TBP = TB // 8
        eaw = ea_ref[...]
        xjw = xj_ref[...]
        eav = jnp.concatenate([eaw[:, j * EED:(j + 1) * EED] for j in range(8)], axis=0)
        xjv = jnp.concatenate([xjw[:, j * D:(j + 1) * D] for j in range(8)], axis=0)
        h1 = jnp.maximum(jnp.dot(eav, w1_ref[...], preferred_element_type=f32)
                         + b1_ref[...], 0.0).astype(bf)
        h2 = jnp.maximum(jnp.dot(h1, w2_ref[...], preferred_element_type=f32)
                         + b2_ref[...], 0.0).astype(bf)
        ee = jnp.dot(h2, w3_ref[...], preferred_element_type=f32).astype(bf)
        xb = xjv.astype(bf)
        xr = jnp.dot(xb, R_ref[...], preferred_element_type=f32).astype(bf)
        msg = (jnp.dot(ee * xr, S_ref[...], preferred_element_type=f32)
               + jnp.dot(xb, B3_ref[...], preferred_element_type=f32))
        o_ref[...] = jnp.concatenate(
            [msg[j * TBP:(j + 1) * TBP, :] for j in range(8)], axis=1)

    return pl.pallas_call(
        body,
        grid=(E // TB,),
        in_specs=[
            pl.BlockSpec((TB // 8, 8 * EED), lambda i: (i, 0)),
            pl.BlockSpec((TB // 8, 8 * D), lambda i: (i, 0)),
            _full((EED, EHID)), _full((1, EHID)),
            _full((EHID, EHID)), _full((1, EHID)),
            _full((EHID, D * D)), _full((D, D)),
            _full((D, D * D)), _full((D * D, D)),
        ],
        out_specs=pl.BlockSpec((TB // 8, 8 * D), lambda i: (i, 0)),
        out_shape=jax.ShapeDtypeStruct((E // 8, 8 * D), jnp.float32),
    )(edge_attr, xj, w1, b1, w2, b2, w3, B3m, Rm, Sm)


def _gru(parts, h, wir, wiz, win, whr, whz, whn, bir, biz, bin_, bhr, bhz, bhn):
    TB = 2000

    def body(p_ref, h_ref, wir_ref, wiz_ref, win_ref, whr_ref, whz_ref, whn_ref,
             bir_ref, biz_ref, bin_ref, bhr_ref, bhz_ref, bhn_ref, o_ref):
        f32 = jnp.float32
        aggr = p_ref[0] + p_ref[1]
        hv = h_ref[...]
        r = jax.nn.sigmoid(jnp.dot(aggr, wir_ref[...], preferred_element_type=f32) + bir_ref[...]
                           + jnp.dot(hv, whr_ref[...], preferred_element_type=f32) + bhr_ref[...])
        z = jax.nn.sigmoid(jnp.dot(aggr, wiz_ref[...], preferred_element_type=f32) + biz_ref[...]
                           + jnp.dot(hv, whz_ref[...], preferred_element_type=f32) + bhz_ref[...])
        n = jnp.tanh(jnp.dot(aggr, win_ref[...], preferred_element_type=f32) + bin_ref[...]
                     + r * (jnp.dot(hv, whn_ref[...], preferred_element_type=f32) + bhn_ref[...]))
        o_ref[...] = (1.0 - z) * n + z * hv

    wspec = _full((D, D))
    bspec = _full((1, D))
    return pl.pallas_call(
        body,
        grid=(N // TB,),
        in_specs=[
            pl.BlockSpec((NC, TB, D), lambda i: (0, i, 0)),
            pl.BlockSpec((TB, D), lambda i: (i, 0)),
            wspec, wspec, wspec, wspec, wspec, wspec,
            bspec, bspec, bspec, bspec, bspec, bspec,
        ],
        out_specs=pl.BlockSpec((TB, D), lambda i: (i, 0)),
        out_shape=jax.ShapeDtypeStruct((N, D), jnp.float32),
    )(parts, h, wir, wiz, win, whr, whz, whn, bir, biz, bin_, bhr, bhz, bhn)


def _readout(h, node, batch2, wi0, bi0, wi1, bi1, wi2, bi2,
             wj0, bj0, wj1, bj1, wj2, bj2, wo0, bo0, wo1, bo1):
    TB = 2000

    def body(h_ref, node_ref, batch_ref, wi0_ref, bi0_ref, wi1_ref, bi1_ref,
             wi2_ref, bi2_ref, wj0_ref, bj0_ref, wj1_ref, bj1_ref, wj2_ref,
             bj2_ref, wo0_ref, bo0_ref, wo1_ref, bo1_ref, o_ref, acc_ref):
        f32 = jnp.float32
        i = pl.program_id(0)

        @pl.when(i == 0)
        def _init():
            acc_ref[...] = jnp.zeros_like(acc_ref)

        hv = h_ref[...]
        hf = jnp.concatenate([hv, node_ref[...]], axis=1)
        a = jnp.maximum(jnp.dot(hf, wi0_ref[...], preferred_element_type=f32) + bi0_ref[...], 0.0)
        a = jnp.maximum(jnp.dot(a, wi1_ref[...], preferred_element_type=f32) + bi1_ref[...], 0.0)
        logits = jnp.dot(a, wi2_ref[...], preferred_element_type=f32) + bi2_ref[...]
        m = jnp.max(logits, axis=1, keepdims=True)
        ex = jnp.exp(logits - m)
        act = ex / jnp.sum(ex, axis=1, keepdims=True)
        b = jnp.maximum(jnp.dot(hv, wj0_ref[...], preferred_element_type=f32) + bj0_ref[...], 0.0)
        b = jnp.maximum(jnp.dot(b, wj1_ref[...], preferred_element_type=f32) + bj1_ref[...], 0.0)
        jv = jnp.dot(b, wj2_ref[...], preferred_element_type=f32) + bj2_ref[...]
        mult = act * jv
        mask = (batch_ref[...] == lax.broadcasted_iota(jnp.int32, (TB, G), 1)).astype(f32)
        acc_ref[...] += lax.dot_general(mask, mult, (((0,), (0,)), ((), ())),
                                        preferred_element_type=f32)

        @pl.when(i == pl.num_programs(0) - 1)
        def _fin():
            Racc = acc_ref[...]
            o = jnp.maximum(jnp.dot(Racc, wo0_ref[...], preferred_element_type=f32)
                            + bo0_ref[...], 0.0)
            o = jnp.dot(o, wo1_ref[...], preferred_element_type=f32) + bo1_ref[...]
            mo = jnp.max(o, axis=1, keepdims=True)
            lse = mo + jnp.log(jnp.sum(jnp.exp(o - mo), axis=1, keepdims=True))
            o_ref[...] = o - lse

    return pl.pallas_call(
        body,
        grid=(N // TB,),
        in_specs=[
            pl.BlockSpec((TB, D), lambda i: (i, 0)),
            pl.BlockSpec((TB, D), lambda i: (i, 0)),
            pl.BlockSpec((TB, 1), lambda i: (i, 0)),
            _full((2 * D, 2 * D)), _full((1, 2 * D)),
            _full((2 * D, 2 * D)), _full((1, 2 * D)),
            _full((2 * D, D)), _full((1, D)),
            _full((D, 2 * D)), _full((1, 2 * D)),
            _full((2 * D, 2 * D)), _full((1, 2 * D)),
            _full((2 * D, D)), _full((1, D)),
            _full((D, 2 * D)), _full((1, 2 * D)),
            _full((2 * D, 2)), _full((1, 2)),
        ],
        out_specs=pl.BlockSpec((G, 2), lambda i: (0, 0)),
        out_shape=jax.ShapeDtypeStruct((G, 2), jnp.float32),
        scratch_shapes=[pltpu.VMEM((G, D), jnp.float32)],
    )(h, node, batch2, wi0, bi0, wi1, bi1, wi2, bi2,
      wj0, bj0, wj1, bj1, wj2, bj2, wo0, bo0, wo1, bo1)


_RM = np.kron(np.ones((1, D), np.float32), np.eye(D, dtype=np.float32))
_SM = np.kron(np.eye(D, dtype=np.float32), np.ones((D, 1), np.float32))


def kernel(x, edge_attr, edge_index, batch, params):
    src2 = edge_index[0].reshape(E // CH, CH)
    dst2 = edge_index[1].reshape(E // CH, CH)
    batch2 = batch.reshape(N, 1)
    zeros_nd = jnp.zeros((N, D), jnp.float32)
    Rm = jnp.asarray(_RM, jnp.bfloat16)
    Sm = jnp.asarray(_SM, jnp.bfloat16)
    ea16 = edge_attr.astype(jnp.bfloat16).reshape(E // 8, 8 * EED)

    node = _node_emb(x, params["node_emb"]["w"], params["node_emb"]["b"].reshape(1, D))
    h = node
    we = params["edge_emb"]["w"]
    be = params["edge_emb"]["b"]
    bf = jnp.bfloat16
    for t in range(STEPS):
        lay = params["layers"][t]
        en = lay["edge_net"]
        xj = _sc_gather(h, src2)
        w1f = (we @ en[0]["w"]).astype(bf)
        b1f = (be @ en[0]["w"] + en[0]["b"]).reshape(1, -1)
        B3m = en[2]["b"].reshape(D, D).T.astype(bf)
        msgw = _msg(ea16, xj.reshape(E // 8, 8 * D),
                    w1f, b1f,
                    en[1]["w"].astype(bf), en[1]["b"].reshape(1, -1).astype(bf),
                    en[2]["w"].astype(bf), B3m, Rm, Sm)
        parts = _sc_scatter(msgw.reshape(E, D), dst2, zeros_nd)
        gru = lay["gru"]
        wi, wh, bi, bh = gru["wi"], gru["wh"], gru["bi"], gru["bh"]
        h = _gru(parts, h,
                 wi[:D].T, wi[D:2 * D].T, wi[2 * D:].T,
                 wh[:D].T, wh[D:2 * D].T, wh[2 * D:].T,
                 bi[:D].reshape(1, D), bi[D:2 * D].reshape(1, D), bi[2 * D:].reshape(1, D),
                 bh[:D].reshape(1, D), bh[D:2 * D].reshape(1, D), bh[2 * D:].reshape(1, D))

    pi, pj, po = params["i"], params["j"], params["out"]
    return _readout(
        h, node, batch2,
        pi[0]["w"], pi[0]["b"].reshape(1, -1),
        pi[1]["w"], pi[1]["b"].reshape(1, -1),
        pi[2]["w"], pi[2]["b"].reshape(1, -1),
        pj[0]["w"], pj[0]["b"].reshape(1, -1),
        pj[1]["w"], pj[1]["b"].reshape(1, -1),
        pj[2]["w"], pj[2]["b"].reshape(1, -1),
        po[0]["w"], po[0]["b"].reshape(1, -1),
        po[1]["w"], po[1]["b"].reshape(1, -1))


# speedup vs baseline: 1.7410x; 1.0067x over previous
import functools

import jax
import jax.numpy as jnp
import numpy as np
from jax import lax
from jax.experimental import pallas as pl
from jax.experimental.pallas import tpu as pltpu
from jax.experimental.pallas import tpu_sc as plsc

N = 10000
E = 160000
D = 16
EED = 16
EHID = 64
STEPS = 3
G = 64

NC = 2
NS = 16
NW = NC * NS
EPW = E // NW
CH = 125
NCHUNK = EPW // CH
KFIRE = 20
NPS = N // NS

_SC_MESH = plsc.VectorSubcoreMesh(
    core_axis_name="c", subcore_axis_name="s", num_cores=NC, num_subcores=NS)
_SC_PARAMS = pltpu.CompilerParams(use_tc_tiling_on_sc=False)


@functools.partial(
    pl.kernel,
    out_type=jax.ShapeDtypeStruct((E, D), jnp.float32),
    mesh=_SC_MESH,
    compiler_params=_SC_PARAMS,
    scratch_types=[
        pltpu.VMEM((NCHUNK, CH), jnp.int32),
        pltpu.VMEM((EPW, D), jnp.float32),
        pltpu.SemaphoreType.DMA,
    ],
)
def _sc_gather(h_hbm, src_hbm, out_hbm, idx_v, rows_v, sem):
    wid = lax.axis_index("s") * NC + lax.axis_index("c")
    pltpu.sync_copy(src_hbm.at[pl.ds(wid * NCHUNK, NCHUNK)], idx_v)

    def group(go, carry):
        handles = []
        for b in range(KFIRE):
            g = go * KFIRE + b
            handles.append(pltpu.async_copy(
                h_hbm.at[idx_v.at[g]], rows_v.at[pl.ds(g * CH, CH)], sem))
        for h in handles:
            h.wait()
        return carry

    lax.fori_loop(0, NCHUNK // KFIRE, group, 0)
    pltpu.sync_copy(rows_v, out_hbm.at[pl.ds(wid * EPW, EPW)])


@functools.partial(
    pl.kernel,
    out_type=jax.ShapeDtypeStruct((NC, N, D), jnp.float32),
    mesh=_SC_MESH,
    compiler_params=_SC_PARAMS,
    scratch_types=[
        pltpu.VMEM((NCHUNK, CH), jnp.int32),
        pltpu.VMEM((EPW, D), jnp.float32),
        pltpu.VMEM_SHARED((N, D), jnp.float32),
        pltpu.SemaphoreType.DMA,
    ],
)
def _sc_scatter(msg_hbm, dst_hbm, zeros_hbm, out_hbm, idx_v, rows_v, acc_sh, sem):
    c = lax.axis_index("c")
    s = lax.axis_index("s")
    wid = s * NC + c
    pltpu.sync_copy(zeros_hbm.at[pl.ds(s * NPS, NPS)], acc_sh.at[pl.ds(s * NPS, NPS)])
    pltpu.sync_copy(dst_hbm.at[pl.ds(wid * NCHUNK, NCHUNK)], idx_v)
    pltpu.sync_copy(msg_hbm.at[pl.ds(wid * EPW, EPW)], rows_v)
    plsc.subcore_barrier()

    def group(go, carry):
        handles = []
        for b in range(KFIRE):
            g = go * KFIRE + b
            handles.append(pltpu.async_copy(
                rows_v.at[pl.ds(g * CH, CH)], acc_sh.at[idx_v.at[g]], sem, add=True))
        for h in handles:
            h.wait()
        return carry

    lax.fori_loop(0, NCHUNK // KFIRE, group, 0)
    plsc.subcore_barrier()
    pltpu.sync_copy(acc_sh.at[pl.ds(s * NPS, NPS)], out_hbm.at[c, pl.ds(s * NPS, NPS)])


def _full(shape):
    return pl.BlockSpec(shape, lambda i: (0,) * len(shape))


def _node_emb(x, w, b):
    TB = 2000

    def body(x_ref, w_ref, b_ref, o_ref):
        o_ref[...] = jnp.dot(x_ref[...], w_ref[...],
                             preferred_element_type=jnp.float32) + b_ref[...]

    return pl.pallas_call(
        body,
        grid=(N // TB,),
        in_specs=[
            pl.BlockSpec((TB, 128), lambda i: (i, 0)),
            _full((128, D)),
            _full((1, D)),
        ],
        out_specs=pl.BlockSpec((TB, D), lambda i: (i, 0)),
        out_shape=jax.ShapeDtypeStruct((N, D), jnp.float32),
    )(x, w, b)


def _msg(edge_attr, xj, w1, b1, w2, b2, w3, B3m, Rm, Sm):
    TB = 6400

    def body(ea_ref, xj_ref, w1_ref, b1_ref, w2_ref, b2_ref,
             w3_ref, B3_ref, R_ref, S_ref, o_ref):
        f32 = jnp.float32
        bf = jnp.bfloat16
        TBP = TB // 8
        eaw = ea_ref[...]
        xjw = xj_ref[...]
        xjv = jnp.concatenate([xjw[:, j * D:(j + 1) * D] for j in range(8)], axis=0)
        h1p = jnp.maximum(jnp.dot(eaw, w1_ref[...], preferred_element_type=f32)
                          + b1_ref[...], 0.0).astype(bf)
        h2p = jnp.maximum(jnp.dot(h1p, w2_ref[...], preferred_element_type=f32)
                          + b2_ref[...], 0.0).astype(bf)
        h2 = jnp.concatenate(
            [h2p[:, j * EHID:(j + 1) * EHID] for j in range(8)], axis=0)
        ee = jnp.dot(h2, w3_ref[...], preferred_element_type=f32).astype(bf)
        xb = xjv.astype(bf)
        xr = jnp.dot(xb, R_ref[...], preferred_element_type=f32).astype(bf)
        msg = (jnp.dot(ee * xr, S_ref[...], preferred_element_type=f32)
               + jnp.dot(xb, B3_ref[...], preferred_element_type=f32))
        o_ref[...] = jnp.concatenate(
            [msg[j * TBP:(j + 1) * TBP, :] for j in range(8)], axis=1)

    return pl.pallas_call(
        body,
        grid=(E // TB,),
        in_specs=[
            pl.BlockSpec((TB // 8, 8 * EED), lambda i: (i, 0)),
            pl.BlockSpec((TB // 8, 8 * D), lambda i: (i, 0)),
            _full((8 * EED, 8 * EHID)), _full((1, 8 * EHID)),
            _full((8 * EHID, 8 * EHID)), _full((1, 8 * EHID)),
            _full((EHID, D * D)), _full((D, D)),
            _full((D, D * D)), _full((D * D, D)),
        ],
        out_specs=pl.BlockSpec((TB // 8, 8 * D), lambda i: (i, 0)),
        out_shape=jax.ShapeDtypeStruct((E // 8, 8 * D), jnp.float32),
    )(edge_attr, xj, w1, b1, w2, b2, w3, B3m, Rm, Sm)


def _gru(parts, h, wir, wiz, win, whr, whz, whn, bir, biz, bin_, bhr, bhz, bhn):
    TB = 2000

    def body(p_ref, h_ref, wir_ref, wiz_ref, win_ref, whr_ref, whz_ref, whn_ref,
             bir_ref, biz_ref, bin_ref, bhr_ref, bhz_ref, bhn_ref, o_ref):
        f32 = jnp.float32
        aggr = p_ref[0] + p_ref[1]
        hv = h_ref[...]
        r = jax.nn.sigmoid(jnp.dot(aggr, wir_ref[...], preferred_element_type=f32) + bir_ref[...]
                           + jnp.dot(hv, whr_ref[...], preferred_element_type=f32) + bhr_ref[...])
        z = jax.nn.sigmoid(jnp.dot(aggr, wiz_ref[...], preferred_element_type=f32) + biz_ref[...]
                           + jnp.dot(hv, whz_ref[...], preferred_element_type=f32) + bhz_ref[...])
        n = jnp.tanh(jnp.dot(aggr, win_ref[...], preferred_element_type=f32) + bin_ref[...]
                     + r * (jnp.dot(hv, whn_ref[...], preferred_element_type=f32) + bhn_ref[...]))
        o_ref[...] = (1.0 - z) * n + z * hv

    wspec = _full((D, D))
    bspec = _full((1, D))
    return pl.pallas_call(
        body,
        grid=(N // TB,),
        in_specs=[
            pl.BlockSpec((NC, TB, D), lambda i: (0, i, 0)),
            pl.BlockSpec((TB, D), lambda i: (i, 0)),
            wspec, wspec, wspec, wspec, wspec, wspec,
            bspec, bspec, bspec, bspec, bspec, bspec,
        ],
        out_specs=pl.BlockSpec((TB, D), lambda i: (i, 0)),
        out_shape=jax.ShapeDtypeStruct((N, D), jnp.float32),
    )(parts, h, wir, wiz, win, whr, whz, whn, bir, biz, bin_, bhr, bhz, bhn)


def _readout(h, node, batch2, wi0, bi0, wi1, bi1, wi2, bi2,
             wj0, bj0, wj1, bj1, wj2, bj2, wo0, bo0, wo1, bo1):
    TB = 2000

    def body(h_ref, node_ref, batch_ref, wi0_ref, bi0_ref, wi1_ref, bi1_ref,
             wi2_ref, bi2_ref, wj0_ref, bj0_ref, wj1_ref, bj1_ref, wj2_ref,
             bj2_ref, wo0_ref, bo0_ref, wo1_ref, bo1_ref, o_ref, acc_ref):
        f32 = jnp.float32
        i = pl.program_id(0)

        @pl.when(i == 0)
        def _init():
            acc_ref[...] = jnp.zeros_like(acc_ref)

        hv = h_ref[...]
        hf = jnp.concatenate([hv, node_ref[...]], axis=1)
        a = jnp.maximum(jnp.dot(hf, wi0_ref[...], preferred_element_type=f32) + bi0_ref[...], 0.0)
        a = jnp.maximum(jnp.dot(a, wi1_ref[...], preferred_element_type=f32) + bi1_ref[...], 0.0)
        logits = jnp.dot(a, wi2_ref[...], preferred_element_type=f32) + bi2_ref[...]
        m = jnp.max(logits, axis=1, keepdims=True)
        ex = jnp.exp(logits - m)
        act = ex / jnp.sum(ex, axis=1, keepdims=True)
        b = jnp.maximum(jnp.dot(hv, wj0_ref[...], preferred_element_type=f32) + bj0_ref[...], 0.0)
        b = jnp.maximum(jnp.dot(b, wj1_ref[...], preferred_element_type=f32) + bj1_ref[...], 0.0)
        jv = jnp.dot(b, wj2_ref[...], preferred_element_type=f32) + bj2_ref[...]
        mult = act * jv
        mask = (batch_ref[...] == lax.broadcasted_iota(jnp.int32, (TB, G), 1)).astype(f32)
        acc_ref[...] += lax.dot_general(mask, mult, (((0,), (0,)), ((), ())),
                                        preferred_element_type=f32)

        @pl.when(i == pl.num_programs(0) - 1)
        def _fin():
            Racc = acc_ref[...]
            o = jnp.maximum(jnp.dot(Racc, wo0_ref[...], preferred_element_type=f32)
                            + bo0_ref[...], 0.0)
            o = jnp.dot(o, wo1_ref[...], preferred_element_type=f32) + bo1_ref[...]
            mo = jnp.max(o, axis=1, keepdims=True)
            lse = mo + jnp.log(jnp.sum(jnp.exp(o - mo), axis=1, keepdims=True))
            o_ref[...] = o - lse

    return pl.pallas_call(
        body,
        grid=(N // TB,),
        in_specs=[
            pl.BlockSpec((TB, D), lambda i: (i, 0)),
            pl.BlockSpec((TB, D), lambda i: (i, 0)),
            pl.BlockSpec((TB, 1), lambda i: (i, 0)),
            _full((2 * D, 2 * D)), _full((1, 2 * D)),
            _full((2 * D, 2 * D)), _full((1, 2 * D)),
            _full((2 * D, D)), _full((1, D)),
            _full((D, 2 * D)), _full((1, 2 * D)),
            _full((2 * D, 2 * D)), _full((1, 2 * D)),
            _full((2 * D, D)), _full((1, D)),
            _full((D, 2 * D)), _full((1, 2 * D)),
            _full((2 * D, 2)), _full((1, 2)),
        ],
        out_specs=pl.BlockSpec((G, 2), lambda i: (0, 0)),
        out_shape=jax.ShapeDtypeStruct((G, 2), jnp.float32),
        scratch_shapes=[pltpu.VMEM((G, D), jnp.float32)],
    )(h, node, batch2, wi0, bi0, wi1, bi1, wi2, bi2,
      wj0, bj0, wj1, bj1, wj2, bj2, wo0, bo0, wo1, bo1)


_RM = np.kron(np.ones((1, D), np.float32), np.eye(D, dtype=np.float32))
_SM = np.kron(np.eye(D, dtype=np.float32), np.ones((D, 1), np.float32))


def kernel(x, edge_attr, edge_index, batch, params):
    src2 = edge_index[0].reshape(E // CH, CH)
    dst2 = edge_index[1].reshape(E // CH, CH)
    batch2 = batch.reshape(N, 1)
    zeros_nd = jnp.zeros((N, D), jnp.float32)
    Rm = jnp.asarray(_RM, jnp.bfloat16)
    Sm = jnp.asarray(_SM, jnp.bfloat16)
    ea16 = edge_attr.astype(jnp.bfloat16).reshape(E // 8, 8 * EED)

    node = _node_emb(x, params["node_emb"]["w"], params["node_emb"]["b"].reshape(1, D))
    h = node
    we = params["edge_emb"]["w"]
    be = params["edge_emb"]["b"]
    bf = jnp.bfloat16
    for t in range(STEPS):
        lay = params["layers"][t]
        en = lay["edge_net"]
        xj = _sc_gather(h, src2)
        eye8 = jnp.eye(8, dtype=jnp.float32)
        w1f = we @ en[0]["w"]
        b1f = (be @ en[0]["w"] + en[0]["b"]).reshape(1, -1)
        w1bd = jnp.kron(eye8, w1f).astype(bf)
        b1bd = jnp.tile(b1f, (1, 8))
        w2bd = jnp.kron(eye8, en[1]["w"]).astype(bf)
        b2bd = jnp.tile(en[1]["b"].reshape(1, -1), (1, 8)).astype(bf)
        B3m = en[2]["b"].reshape(D, D).T.astype(bf)
        msgw = _msg(ea16, xj.reshape(E // 8, 8 * D),
                    w1bd, b1bd, w2bd, b2bd,
                    en[2]["w"].astype(bf), B3m, Rm, Sm)
        parts = _sc_scatter(msgw.reshape(E, D), dst2, zeros_nd)
        gru = lay["gru"]
        wi, wh, bi, bh = gru["wi"], gru["wh"], gru["bi"], gru["bh"]
        h = _gru(parts, h,
                 wi[:D].T, wi[D:2 * D].T, wi[2 * D:].T,
                 wh[:D].T, wh[D:2 * D].T, wh[2 * D:].T,
                 bi[:D].reshape(1, D), bi[D:2 * D].reshape(1, D), bi[2 * D:].reshape(1, D),
                 bh[:D].reshape(1, D), bh[D:2 * D].reshape(1, D), bh[2 * D:].reshape(1, D))

    pi, pj, po = params["i"], params["j"], params["out"]
    return _readout(
        h, node, batch2,
        pi[0]["w"], pi[0]["b"].reshape(1, -1),
        pi[1]["w"], pi[1]["b"].reshape(1, -1),
        pi[2]["w"], pi[2]["b"].reshape(1, -1),
        pj[0]["w"], pj[0]["b"].reshape(1, -1),
        pj[1]["w"], pj[1]["b"].reshape(1, -1),
        pj[2]["w"], pj[2]["b"].reshape(1, -1),
        po[0]["w"], po[0]["b"].reshape(1, -1),
        po[1]["w"], po[1]["b"].reshape(1, -1))


# speedup vs baseline: 1.7590x; 1.0103x over previous
import functools

import jax
import jax.numpy as jnp
import numpy as np
from jax import lax
from jax.experimental import pallas as pl
from jax.experimental.pallas import tpu as pltpu
from jax.experimental.pallas import tpu_sc as plsc

N = 10000
E = 160000
D = 16
EED = 16
EHID = 64
STEPS = 3
G = 64

NC = 2
NS = 16
NW = NC * NS
EPW = E // NW
CH = 125
NCHUNK = EPW // CH
KFIRE = 20
NPS = N // NS

_SC_MESH = plsc.VectorSubcoreMesh(
    core_axis_name="c", subcore_axis_name="s", num_cores=NC, num_subcores=NS)
_SC_PARAMS = pltpu.CompilerParams(use_tc_tiling_on_sc=False)


@functools.partial(
    pl.kernel,
    out_type=jax.ShapeDtypeStruct((E, D), jnp.float32),
    mesh=_SC_MESH,
    compiler_params=_SC_PARAMS,
    scratch_types=[
        pltpu.VMEM((NCHUNK, CH), jnp.int32),
        pltpu.VMEM((EPW, D), jnp.float32),
        pltpu.SemaphoreType.DMA,
    ],
)
def _sc_gather(h_hbm, src_hbm, out_hbm, idx_v, rows_v, sem):
    wid = lax.axis_index("s") * NC + lax.axis_index("c")
    pltpu.sync_copy(src_hbm.at[pl.ds(wid * NCHUNK, NCHUNK)], idx_v)

    def group(go, carry):
        handles = []
        for b in range(KFIRE):
            g = go * KFIRE + b
            handles.append(pltpu.async_copy(
                h_hbm.at[idx_v.at[g]], rows_v.at[pl.ds(g * CH, CH)], sem))
        for h in handles:
            h.wait()
        return carry

    lax.fori_loop(0, NCHUNK // KFIRE, group, 0)
    pltpu.sync_copy(rows_v, out_hbm.at[pl.ds(wid * EPW, EPW)])


@functools.partial(
    pl.kernel,
    out_type=jax.ShapeDtypeStruct((NC, N, D), jnp.float32),
    mesh=_SC_MESH,
    compiler_params=_SC_PARAMS,
    scratch_types=[
        pltpu.VMEM((NCHUNK, CH), jnp.int32),
        pltpu.VMEM((EPW, D), jnp.float32),
        pltpu.VMEM_SHARED((N, D), jnp.float32),
        pltpu.SemaphoreType.DMA,
    ],
)
def _sc_scatter(msg_hbm, dst_hbm, zeros_hbm, out_hbm, idx_v, rows_v, acc_sh, sem):
    c = lax.axis_index("c")
    s = lax.axis_index("s")
    wid = s * NC + c
    pltpu.sync_copy(zeros_hbm.at[pl.ds(s * NPS, NPS)], acc_sh.at[pl.ds(s * NPS, NPS)])
    pltpu.sync_copy(dst_hbm.at[pl.ds(wid * NCHUNK, NCHUNK)], idx_v)
    pltpu.sync_copy(msg_hbm.at[pl.ds(wid * EPW, EPW)], rows_v)
    plsc.subcore_barrier()

    def group(go, carry):
        handles = []
        for b in range(KFIRE):
            g = go * KFIRE + b
            handles.append(pltpu.async_copy(
                rows_v.at[pl.ds(g * CH, CH)], acc_sh.at[idx_v.at[g]], sem, add=True))
        for h in handles:
            h.wait()
        return carry

    lax.fori_loop(0, NCHUNK // KFIRE, group, 0)
    plsc.subcore_barrier()
    pltpu.sync_copy(acc_sh.at[pl.ds(s * NPS, NPS)], out_hbm.at[c, pl.ds(s * NPS, NPS)])


def _full(shape):
    return pl.BlockSpec(shape, lambda i: (0,) * len(shape))


def _node_emb(x, w, b):
    TB = 2000

    def body(x_ref, w_ref, b_ref, o_ref):
        o_ref[...] = jnp.dot(x_ref[...], w_ref[...],
                             preferred_element_type=jnp.float32) + b_ref[...]

    return pl.pallas_call(
        body,
        grid=(N // TB,),
        in_specs=[
            pl.BlockSpec((TB, 128), lambda i: (i, 0)),
            _full((128, D)),
            _full((1, D)),
        ],
        out_specs=pl.BlockSpec((TB, D), lambda i: (i, 0)),
        out_shape=jax.ShapeDtypeStruct((N, D), jnp.float32),
    )(x, w, b)


def _msg(edge_attr, xj, w1, b1, w2, b2, w3, B3m, Rm, Sm):
    TB = 6400

    def body(ea_ref, xj_ref, w1_ref, b1_ref, w2_ref, b2_ref,
             w3_ref, B3_ref, R_ref, S_ref, o_ref):
        f32 = jnp.float32
        bf = jnp.bfloat16
        TBP = TB // 8
        eaw = ea_ref[...]
        xjw = xj_ref[...]
        xjv = jnp.concatenate([xjw[:, j * D:(j + 1) * D] for j in range(8)], axis=0)
        h1p = jnp.maximum(jnp.dot(eaw, w1_ref[...], preferred_element_type=f32)
                          + b1_ref[...], 0.0).astype(bf)
        h2p = jnp.maximum(jnp.dot(h1p, w2_ref[...], preferred_element_type=f32)
                          + b2_ref[...], 0.0).astype(bf)
        h2 = jnp.concatenate(
            [h2p[:, j * EHID:(j + 1) * EHID] for j in range(8)], axis=0)
        ee = jnp.dot(h2, w3_ref[...], preferred_element_type=f32).astype(bf)
        xb = xjv.astype(bf)
        xr = jnp.dot(xb, R_ref[...], preferred_element_type=f32).astype(bf)
        msg = (jnp.dot(ee * xr, S_ref[...], preferred_element_type=f32)
               + jnp.dot(xb, B3_ref[...], preferred_element_type=f32))
        o_ref[...] = jnp.concatenate(
            [msg[j * TBP:(j + 1) * TBP, :] for j in range(8)], axis=1)

    return pl.pallas_call(
        body,
        grid=(E // TB,),
        in_specs=[
            pl.BlockSpec((TB // 8, 8 * EED), lambda i: (i, 0)),
            pl.BlockSpec((TB // 8, 8 * D), lambda i: (i, 0)),
            _full((8 * EED, 8 * EHID)), _full((1, 8 * EHID)),
            _full((8 * EHID, 8 * EHID)), _full((1, 8 * EHID)),
            _full((EHID, D * D)), _full((D, D)),
            _full((D, D * D)), _full((D * D, D)),
        ],
        out_specs=pl.BlockSpec((TB // 8, 8 * D), lambda i: (i, 0)),
        out_shape=jax.ShapeDtypeStruct((E // 8, 8 * D), jnp.float32),
    )(edge_attr, xj, w1, b1, w2, b2, w3, B3m, Rm, Sm)


def _gru(parts, h, wir, wiz, win, whr, whz, whn, bir, biz, bin_, bhr, bhz, bhn):
    TB = 2000

    def body(p_ref, h_ref, wir_ref, wiz_ref, win_ref, whr_ref, whz_ref, whn_ref,
             bir_ref, biz_ref, bin_ref, bhr_ref, bhz_ref, bhn_ref, o_ref):
        f32 = jnp.float32
        aggr = p_ref[0] + p_ref[1]
        hv = h_ref[...]
        r = jax.nn.sigmoid(jnp.dot(aggr, wir_ref[...], preferred_element_type=f32) + bir_ref[...]
                           + jnp.dot(hv, whr_ref[...], preferred_element_type=f32) + bhr_ref[...])
        z = jax.nn.sigmoid(jnp.dot(aggr, wiz_ref[...], preferred_element_type=f32) + biz_ref[...]
                           + jnp.dot(hv, whz_ref[...], preferred_element_type=f32) + bhz_ref[...])
        n = jnp.tanh(jnp.dot(aggr, win_ref[...], preferred_element_type=f32) + bin_ref[...]
                     + r * (jnp.dot(hv, whn_ref[...], preferred_element_type=f32) + bhn_ref[...]))
        o_ref[...] = (1.0 - z) * n + z * hv

    wspec = _full((D, D))
    bspec = _full((1, D))
    return pl.pallas_call(
        body,
        grid=(N // TB,),
        in_specs=[
            pl.BlockSpec((NC, TB, D), lambda i: (0, i, 0)),
            pl.BlockSpec((TB, D), lambda i: (i, 0)),
            wspec, wspec, wspec, wspec, wspec, wspec,
            bspec, bspec, bspec, bspec, bspec, bspec,
        ],
        out_specs=pl.BlockSpec((TB, D), lambda i: (i, 0)),
        out_shape=jax.ShapeDtypeStruct((N, D), jnp.float32),
    )(parts, h, wir, wiz, win, whr, whz, whn, bir, biz, bin_, bhr, bhz, bhn)


def _readout(parts, h, node, batch2,
             wir, wiz, win, whr, whz, whn, bir, biz, bin_, bhr, bhz, bhn,
             wi0, bi0, wi1, bi1, wi2, bi2,
             wj0, bj0, wj1, bj1, wj2, bj2, wo0, bo0, wo1, bo1):
    TB = 2000

    def body(p_ref, h_ref, node_ref, batch_ref,
             wir_ref, wiz_ref, win_ref, whr_ref, whz_ref, whn_ref,
             bir_ref, biz_ref, bin_ref, bhr_ref, bhz_ref, bhn_ref,
             wi0_ref, bi0_ref, wi1_ref, bi1_ref,
             wi2_ref, bi2_ref, wj0_ref, bj0_ref, wj1_ref, bj1_ref, wj2_ref,
             bj2_ref, wo0_ref, bo0_ref, wo1_ref, bo1_ref, o_ref, acc_ref):
        f32 = jnp.float32
        i = pl.program_id(0)

        @pl.when(i == 0)
        def _init():
            acc_ref[...] = jnp.zeros_like(acc_ref)

        aggr = p_ref[0] + p_ref[1]
        hp = h_ref[...]
        rg = jax.nn.sigmoid(jnp.dot(aggr, wir_ref[...], preferred_element_type=f32) + bir_ref[...]
                            + jnp.dot(hp, whr_ref[...], preferred_element_type=f32) + bhr_ref[...])
        zg = jax.nn.sigmoid(jnp.dot(aggr, wiz_ref[...], preferred_element_type=f32) + biz_ref[...]
                            + jnp.dot(hp, whz_ref[...], preferred_element_type=f32) + bhz_ref[...])
        ng = jnp.tanh(jnp.dot(aggr, win_ref[...], preferred_element_type=f32) + bin_ref[...]
                      + rg * (jnp.dot(hp, whn_ref[...], preferred_element_type=f32) + bhn_ref[...]))
        hv = (1.0 - zg) * ng + zg * hp
        hf = jnp.concatenate([hv, node_ref[...]], axis=1)
        a = jnp.maximum(jnp.dot(hf, wi0_ref[...], preferred_element_type=f32) + bi0_ref[...], 0.0)
        a = jnp.maximum(jnp.dot(a, wi1_ref[...], preferred_element_type=f32) + bi1_ref[...], 0.0)
        logits = jnp.dot(a, wi2_ref[...], preferred_element_type=f32) + bi2_ref[...]
        m = jnp.max(logits, axis=1, keepdims=True)
        ex = jnp.exp(logits - m)
        act = ex / jnp.sum(ex, axis=1, keepdims=True)
        b = jnp.maximum(jnp.dot(hv, wj0_ref[...], preferred_element_type=f32) + bj0_ref[...], 0.0)
        b = jnp.maximum(jnp.dot(b, wj1_ref[...], preferred_element_type=f32) + bj1_ref[...], 0.0)
        jv = jnp.dot(b, wj2_ref[...], preferred_element_type=f32) + bj2_ref[...]
        mult = act * jv
        mask = (batch_ref[...] == lax.broadcasted_iota(jnp.int32, (TB, G), 1)).astype(f32)
        acc_ref[...] += lax.dot_general(mask, mult, (((0,), (0,)), ((), ())),
                                        preferred_element_type=f32)

        @pl.when(i == pl.num_programs(0) - 1)
        def _fin():
            Racc = acc_ref[...]
            o = jnp.maximum(jnp.dot(Racc, wo0_ref[...], preferred_element_type=f32)
                            + bo0_ref[...], 0.0)
            o = jnp.dot(o, wo1_ref[...], preferred_element_type=f32) + bo1_ref[...]
            mo = jnp.max(o, axis=1, keepdims=True)
            lse = mo + jnp.log(jnp.sum(jnp.exp(o - mo), axis=1, keepdims=True))
            o_ref[...] = o - lse

    return pl.pallas_call(
        body,
        grid=(N // TB,),
        in_specs=[
            pl.BlockSpec((NC, TB, D), lambda i: (0, i, 0)),
            pl.BlockSpec((TB, D), lambda i: (i, 0)),
            pl.BlockSpec((TB, D), lambda i: (i, 0)),
            pl.BlockSpec((TB, 1), lambda i: (i, 0)),
            _full((D, D)), _full((D, D)), _full((D, D)),
            _full((D, D)), _full((D, D)), _full((D, D)),
            _full((1, D)), _full((1, D)), _full((1, D)),
            _full((1, D)), _full((1, D)), _full((1, D)),
            _full((2 * D, 2 * D)), _full((1, 2 * D)),
            _full((2 * D, 2 * D)), _full((1, 2 * D)),
            _full((2 * D, D)), _full((1, D)),
            _full((D, 2 * D)), _full((1, 2 * D)),
            _full((2 * D, 2 * D)), _full((1, 2 * D)),
            _full((2 * D, D)), _full((1, D)),
            _full((D, 2 * D)), _full((1, 2 * D)),
            _full((2 * D, 2)), _full((1, 2)),
        ],
        out_specs=pl.BlockSpec((G, 2), lambda i: (0, 0)),
        out_shape=jax.ShapeDtypeStruct((G, 2), jnp.float32),
        scratch_shapes=[pltpu.VMEM((G, D), jnp.float32)],
    )(parts, h, node, batch2,
      wir, wiz, win, whr, whz, whn, bir, biz, bin_, bhr, bhz, bhn,
      wi0, bi0, wi1, bi1, wi2, bi2,
      wj0, bj0, wj1, bj1, wj2, bj2, wo0, bo0, wo1, bo1)


_RM = np.kron(np.ones((1, D), np.float32), np.eye(D, dtype=np.float32))
_SM = np.kron(np.eye(D, dtype=np.float32), np.ones((D, 1), np.float32))


def kernel(x, edge_attr, edge_index, batch, params):
    src2 = edge_index[0].reshape(E // CH, CH)
    dst2 = edge_index[1].reshape(E // CH, CH)
    batch2 = batch.reshape(N, 1)
    zeros_nd = jnp.zeros((N, D), jnp.float32)
    Rm = jnp.asarray(_RM, jnp.bfloat16)
    Sm = jnp.asarray(_SM, jnp.bfloat16)
    ea16 = edge_attr.astype(jnp.bfloat16).reshape(E // 8, 8 * EED)

    node = _node_emb(x, params["node_emb"]["w"], params["node_emb"]["b"].reshape(1, D))
    h = node
    we = params["edge_emb"]["w"]
    be = params["edge_emb"]["b"]
    bf = jnp.bfloat16
    for t in range(STEPS):
        lay = params["layers"][t]
        en = lay["edge_net"]
        xj = _sc_gather(h, src2)
        eye8 = jnp.eye(8, dtype=jnp.float32)
        w1f = we @ en[0]["w"]
        b1f = (be @ en[0]["w"] + en[0]["b"]).reshape(1, -1)
        w1bd = jnp.kron(eye8, w1f).astype(bf)
        b1bd = jnp.tile(b1f, (1, 8))
        w2bd = jnp.kron(eye8, en[1]["w"]).astype(bf)
        b2bd = jnp.tile(en[1]["b"].reshape(1, -1), (1, 8)).astype(bf)
        B3m = en[2]["b"].reshape(D, D).T.astype(bf)
        msgw = _msg(ea16, xj.reshape(E // 8, 8 * D),
                    w1bd, b1bd, w2bd, b2bd,
                    en[2]["w"].astype(bf), B3m, Rm, Sm)
        parts = _sc_scatter(msgw.reshape(E, D), dst2, zeros_nd)
        gru = lay["gru"]
        wi, wh, bi, bh = gru["wi"], gru["wh"], gru["bi"], gru["bh"]
        gru_args = (
            wi[:D].T, wi[D:2 * D].T, wi[2 * D:].T,
            wh[:D].T, wh[D:2 * D].T, wh[2 * D:].T,
            bi[:D].reshape(1, D), bi[D:2 * D].reshape(1, D), bi[2 * D:].reshape(1, D),
            bh[:D].reshape(1, D), bh[D:2 * D].reshape(1, D), bh[2 * D:].reshape(1, D))
        if t < STEPS - 1:
            h = _gru(parts, h, *gru_args)

    pi, pj, po = params["i"], params["j"], params["out"]
    return _readout(
        parts, h, node, batch2, *gru_args,
        pi[0]["w"], pi[0]["b"].reshape(1, -1),
        pi[1]["w"], pi[1]["b"].reshape(1, -1),
        pi[2]["w"], pi[2]["b"].reshape(1, -1),
        pj[0]["w"], pj[0]["b"].reshape(1, -1),
        pj[1]["w"], pj[1]["b"].reshape(1, -1),
        pj[2]["w"], pj[2]["b"].reshape(1, -1),
        po[0]["w"], po[0]["b"].reshape(1, -1),
        po[1]["w"], po[1]["b"].reshape(1, -1))
